# Initial kernel scaffold; baseline (speedup 1.0000x reference)
import jax
import jax.numpy as jnp
from jax.experimental import pallas as pl


def kernel(x, table):
    raise NotImplementedError("write your pallas kernel here")


# speedup vs baseline: 23.4078x; 23.4078x over previous
import jax
import jax.numpy as jnp
from jax.experimental import pallas as pl


def _bcast_body(table_ref, out_ref):
    out_ref[...] = jnp.broadcast_to(table_ref[...][None, :, :], out_ref.shape)


def kernel(x, table):
    batch, seq_len = x.shape
    embed_dim = table.shape[1]
    block_b = 32
    grid = (batch // block_b,)
    return pl.pallas_call(
        _bcast_body,
        grid=grid,
        in_specs=[
            pl.BlockSpec((seq_len, embed_dim), lambda i: (0, 0)),
        ],
        out_specs=pl.BlockSpec((block_b, seq_len, embed_dim), lambda i: (i, 0, 0)),
        out_shape=jax.ShapeDtypeStruct((batch, seq_len, embed_dim), table.dtype),
    )(table)
